# baseline (speedup 1.0000x reference)
import functools

import jax
import jax.numpy as jnp
from jax import lax
from jax.experimental import pallas as pl
from jax.experimental.pallas import tpu as pltpu
from jax.experimental.pallas import tpu_sc as plsc

EPS = 1e-5
M_SEG = 500
NC = 2
NS = 16
NW = NC * NS
CPL = 128 // NW

_SC_PARAMS = pltpu.CompilerParams(needs_layout_passes=False)


def _mesh():
    return plsc.VectorSubcoreMesh(core_axis_name="c", subcore_axis_name="s")


def _wid():
    return lax.axis_index("s") * NC + lax.axis_index("c")


def _zero_acc(acc, nwords):
    z = jnp.zeros((16,), jnp.float32)

    @plsc.parallel_loop(0, nwords // 16, unroll=5)
    def _(i):
        for cc in range(CPL):
            acc[cc, pl.ds(i * 16, 16)] = z


def _make_spmm(n, e, chunk):
    nchunk = e // chunk
    groups = chunk // 16

    @functools.partial(
        pl.kernel,
        out_type=jax.ShapeDtypeStruct((128, n), jnp.float32),
        mesh=_mesh(),
        scratch_types=[
            pltpu.VMEM((CPL, n), jnp.float32),
            pltpu.VMEM((CPL, n), jnp.float32),
            pltpu.VMEM((chunk,), jnp.int32),
            pltpu.VMEM((chunk,), jnp.int32),
            pltpu.VMEM((chunk,), jnp.float32),
            pltpu.VMEM((chunk,), jnp.float32),
            pltpu.SemaphoreType.DMA,
            pltpu.SemaphoreType.DMA,
        ],
        compiler_params=_SC_PARAMS,
    )
    def spmm(xt_hbm, rc_hbm, val_hbm, out_hbm,
             xv, acc, rcb0, rcb1, valb0, valb1, sem0, sem1):
        w = _wid()
        pltpu.sync_copy(xt_hbm.at[pl.ds(w * CPL, CPL)], xv)
        _zero_acc(acc, n)
        sems = (sem0, sem1)
        rcbs, valbs = (rcb0, rcb1), (valb0, valb1)

        def start(g, slot):
            s = sems[slot]
            pltpu.async_copy(rc_hbm.at[pl.ds(g * chunk, chunk)], rcbs[slot], s)
            pltpu.async_copy(val_hbm.at[pl.ds(g * chunk, chunk)], valbs[slot], s)

        def wait(slot):
            s = sems[slot]
            pltpu.make_async_copy(rc_hbm.at[pl.ds(0, chunk)], rcbs[slot], s).wait()
            pltpu.make_async_copy(val_hbm.at[pl.ds(0, chunk)], valbs[slot], s).wait()

        def process(slot):
            rcb, vb = rcbs[slot], valbs[slot]

            @plsc.parallel_loop(0, groups, unroll=10)
            def _(i):
                ds = pl.ds(i * 16, 16)
                rc = rcb[ds]
                rvec = lax.shift_right_logical(rc, 14)
                cvec = rc & 16383
                vvec = vb[ds]
                for cc in range(CPL):
                    ci = jnp.full((16,), cc, jnp.int32)
                    g = plsc.load_gather(xv, [ci, cvec])
                    plsc.addupdate_scatter(acc, [ci, rvec], g * vvec)

        start(0, 0)

        def pair(j, _):
            g0 = j * 2
            start(g0 + 1, 1)
            wait(0)
            process(0)

            @pl.when(g0 + 2 < nchunk)
            def _():
                start(g0 + 2, 0)

            wait(1)
            process(1)
            return 0

        lax.fori_loop(0, nchunk // 2, pair, 0)
        pltpu.sync_copy(acc, out_hbm.at[pl.ds(w * CPL, CPL)])

    return spmm


def _make_segsum(n, t):
    @functools.partial(
        pl.kernel,
        out_type=jax.ShapeDtypeStruct((128, t), jnp.float32),
        mesh=_mesh(),
        scratch_types=[
            pltpu.VMEM((CPL, n), jnp.float32),
            pltpu.VMEM((n,), jnp.int32),
            pltpu.VMEM((CPL, t), jnp.float32),
            pltpu.SemaphoreType.DMA,
        ],
        compiler_params=_SC_PARAMS,
    )
    def seg(x_hbm, batch_hbm, out_hbm, xv, bv, acc, sem):
        w = _wid()
        pltpu.async_copy(x_hbm.at[pl.ds(w * CPL, CPL)], xv, sem)
        pltpu.async_copy(batch_hbm, bv, sem)
        _zero_acc(acc, t)
        pltpu.make_async_copy(x_hbm.at[pl.ds(0, CPL)], xv, sem).wait()
        pltpu.make_async_copy(batch_hbm, bv, sem).wait()

        @plsc.parallel_loop(0, n // 16, unroll=8)
        def _(i):
            ds = pl.ds(i * 16, 16)
            bvec = bv[ds]
            for cc in range(CPL):
                ci = jnp.full((16,), cc, jnp.int32)
                plsc.addupdate_scatter(acc, [ci, bvec], xv[cc, ds])
        pltpu.sync_copy(acc, out_hbm.at[pl.ds(w * CPL, CPL)])

    return seg


def _make_gat(n, t):
    @functools.partial(
        pl.kernel,
        out_type=(jax.ShapeDtypeStruct((128, t), jnp.float32),
                  jax.ShapeDtypeStruct((t,), jnp.float32)),
        mesh=_mesh(),
        scratch_types=[
            pltpu.VMEM((CPL, n), jnp.float32),
            pltpu.VMEM((n,), jnp.float32),
            pltpu.VMEM((t,), jnp.float32),
            pltpu.VMEM((n,), jnp.int32),
            pltpu.VMEM((CPL, t), jnp.float32),
            pltpu.VMEM((t,), jnp.float32),
            pltpu.SemaphoreType.DMA,
        ],
        compiler_params=_SC_PARAMS,
    )
    def gat(hs_hbm, asrc_hbm, adst_hbm, batch_hbm, u_hbm, den_hbm,
            xv, asv, adv, bv, accu, accd, sem):
        w = _wid()
        pltpu.async_copy(hs_hbm.at[pl.ds(w * CPL, CPL)], xv, sem)
        pltpu.async_copy(asrc_hbm, asv, sem)
        pltpu.async_copy(adst_hbm, adv, sem)
        pltpu.async_copy(batch_hbm, bv, sem)
        _zero_acc(accu, t)
        z = jnp.zeros((16,), jnp.float32)

        @plsc.parallel_loop(0, t // 16, unroll=5)
        def _(i):
            accd[pl.ds(i * 16, 16)] = z

        pltpu.make_async_copy(hs_hbm.at[pl.ds(0, CPL)], xv, sem).wait()
        pltpu.make_async_copy(asrc_hbm, asv, sem).wait()
        pltpu.make_async_copy(adst_hbm, adv, sem).wait()
        pltpu.make_async_copy(batch_hbm, bv, sem).wait()

        def edge_pass(with_den):
            @plsc.parallel_loop(0, n // 16, unroll=8)
            def _(i):
                ds = pl.ds(i * 16, 16)
                bvec = bv[ds]
                ev = asv[ds] + plsc.load_gather(adv, [bvec])
                ev = jnp.where(ev > 0, ev, 0.01 * ev)
                ex = jnp.exp(ev)
                if with_den:
                    plsc.addupdate_scatter(accd, [bvec], ex)
                for cc in range(CPL):
                    ci = jnp.full((16,), cc, jnp.int32)
                    plsc.addupdate_scatter(accu, [ci, bvec], xv[cc, ds] * ex)

        @pl.when(w == 0)
        def _():
            edge_pass(True)

        @pl.when(w != 0)
        def _():
            edge_pass(False)

        pltpu.sync_copy(accu, u_hbm.at[pl.ds(w * CPL, CPL)])

        @pl.when(w == 0)
        def _():
            pltpu.sync_copy(accd, den_hbm)

    return gat


def _bn_t(h, g, b):
    mu = jnp.mean(h, axis=1, keepdims=True)
    var = jnp.mean((h - mu) ** 2, axis=1, keepdims=True)
    return (h - mu) * lax.rsqrt(var + EPS) * g + b


def _mmT(w, x):
    return lax.dot_general(w, x, (((0,), (0,)), ((), ())),
                           preferred_element_type=jnp.float32)


def _elu(x):
    return jnp.where(x > 0, x, jnp.exp(jnp.where(x > 0, 0.0, x)) - 1.0)


def _gru_t(x, h, wih, whh, bih, bhh):
    gi = _mmT(wih, x) + bih
    gh = _mmT(whh, h) + bhh
    r = jax.nn.sigmoid(gi[0:128] + gh[0:128])
    zz = jax.nn.sigmoid(gi[128:256] + gh[128:256])
    nn_ = jnp.tanh(gi[256:384] + r * gh[256:384])
    return (1.0 - zz) * nn_ + zz * h


def _gcn_update(ax_T, w, b, g, bb):
    n = ax_T.shape[1]

    def body(ax_ref, w_ref, b_ref, g_ref, bb_ref, o_ref):
        h = jnp.maximum(_mmT(w_ref[...], ax_ref[...]) + b_ref[...], 0.0)
        o_ref[...] = _bn_t(h, g_ref[...], bb_ref[...])

    return pl.pallas_call(
        body, out_shape=jax.ShapeDtypeStruct((128, n), jnp.float32),
    )(ax_T, w, b, g, bb)


def _post_gcn2(ax_T, w, b, g, bb, g0, b0, ws, a_s):
    n = ax_T.shape[1]

    def body(ax_ref, w_ref, b_ref, g_ref, bb_ref, g0_ref, b0_ref, ws_ref,
             as_ref, xb0_ref, hs_ref, asrc_ref):
        h = jnp.maximum(_mmT(w_ref[...], ax_ref[...]) + b_ref[...], 0.0)
        x2 = _bn_t(h, g_ref[...], bb_ref[...])
        xb0 = _bn_t(x2, g0_ref[...], b0_ref[...])
        hs = _mmT(ws_ref[...], xb0)
        xb0_ref[...] = xb0
        hs_ref[...] = hs
        asrc_ref[...] = jnp.sum(hs * as_ref[...], axis=0, keepdims=True)

    return pl.pallas_call(
        body,
        out_shape=(jax.ShapeDtypeStruct((128, n), jnp.float32),
                   jax.ShapeDtypeStruct((128, n), jnp.float32),
                   jax.ShapeDtypeStruct((1, n), jnp.float32)),
    )(ax_T, w, b, g, bb, g0, b0, ws, a_s)


def _mid_pool(segx_T, g1, b1, wd, a_d):
    t = segx_T.shape[1]

    def body(sx_ref, g1_ref, b1_ref, wd_ref, ad_ref, mb_ref, adst_ref):
        mean = jnp.maximum(sx_ref[...], 0.0)
        mb = _bn_t(mean, g1_ref[...], b1_ref[...])
        hd = _mmT(wd_ref[...], mb)
        mb_ref[...] = mb
        adst_ref[...] = jnp.sum(hd * ad_ref[...], axis=0, keepdims=True)

    return pl.pallas_call(
        body,
        out_shape=(jax.ShapeDtypeStruct((128, t), jnp.float32),
                   jax.ShapeDtypeStruct((1, t), jnp.float32)),
    )(segx_T, g1, b1, wd, a_d)


def _final(u_T, den, meanb_T, bias_tt, wih_tt, whh_tt, bih_tt, bhh_tt,
           g2_tt, b2_tt, mol, batch2, pred):
    t = u_T.shape[1]
    m = M_SEG

    def body(u_ref, den_ref, mb_ref, bias_ref, wih_ref, whh_ref, bih_ref,
             bhh_ref, g2_ref, b2_ref,
             g0m_ref, b0m_ref, g1m_ref, b1m_ref, g2m_ref, b2m_ref,
             wsm_ref, wdm_ref, asm_ref, adm_ref, biasm_ref,
             wihm_ref, whhm_ref, bihm_ref, bhhm_ref,
             batch_ref, w1_ref, b1p_ref, w2_ref, b2p_ref, o_ref):
        meanb = mb_ref[...]
        gat = u_ref[...] / (den_ref[...] + 1e-16) + bias_ref[...]
        g = _elu(gat)
        gg = _gru_t(g, meanb, wih_ref[...], whh_ref[...], bih_ref[...],
                    bhh_ref[...])
        tt = _bn_t(jnp.maximum(gg, 0.0), g2_ref[...], b2_ref[...])
        xb = _bn_t(tt, g0m_ref[...], b0m_ref[...])
        sel = (lax.broadcasted_iota(jnp.int32, (m, t), 0) ==
               jnp.broadcast_to(batch_ref[...], (m, t))).astype(jnp.float32)
        segx = lax.dot_general(xb, sel, (((1,), (1,)), ((), ())),
                               preferred_element_type=jnp.float32)
        meanm = jnp.maximum(segx, 0.0)
        meanmb = _bn_t(meanm, g1m_ref[...], b1m_ref[...])
        hs = _mmT(wsm_ref[...], xb)
        asrc = jnp.sum(hs * asm_ref[...], axis=0, keepdims=True)
        hd = _mmT(wdm_ref[...], meanmb)
        adst = jnp.sum(hd * adm_ref[...], axis=0, keepdims=True)
        ev = asrc + lax.dot_general(adst, sel, (((1,), (0,)), ((), ())),
                                    preferred_element_type=jnp.float32)
        ev = jnp.where(ev > 0, ev, 0.01 * ev)
        ex = jnp.exp(ev)
        denm = lax.dot_general(ex, sel, (((1,), (1,)), ((), ())),
                               preferred_element_type=jnp.float32)
        um = lax.dot_general(hs * ex, sel, (((1,), (1,)), ((), ())),
                             preferred_element_type=jnp.float32)
        gatm = um / (denm + 1e-16) + biasm_ref[...]
        gm = _elu(gatm)
        ggm = _gru_t(gm, meanmb, wihm_ref[...], whhm_ref[...], bihm_ref[...],
                     bhhm_ref[...])
        molv = _bn_t(jnp.maximum(ggm, 0.0), g2m_ref[...], b2m_ref[...])
        h1 = jnp.maximum(_mmT(w1_ref[...], molv) + b1p_ref[...], 0.0)
        o_ref[...] = _mmT(w2_ref[...], h1) + b2p_ref[...]

    return pl.pallas_call(
        body, out_shape=jax.ShapeDtypeStruct((1, m), jnp.float32),
    )(u_T, den, meanb_T, bias_tt, wih_tt, whh_tt, bih_tt, bhh_tt, g2_tt,
      b2_tt, mol['bn0_g'], mol['bn0_b'], mol['bn1_g'], mol['bn1_b'],
      mol['bn2_g'], mol['bn2_b'], mol['Ws'], mol['Wd'], mol['a_s'],
      mol['a_d'], mol['bias'], mol['Wih'], mol['Whh'], mol['bih'],
      mol['bhh'], batch2, pred['W1'], pred['b1'], pred['W2'], pred['b2'])


def _col(v):
    return v.reshape(-1, 1)


def kernel(node_attr, adj_index, adj_value, tt_node_batch, tt_graph_batch,
           params):
    n, d = node_attr.shape
    e = adj_value.shape[0]
    t = tt_graph_batch.shape[0]
    m = M_SEG

    row = adj_index[0]
    col = adj_index[1]
    x0_T = jnp.swapaxes(node_attr, 0, 1)

    spmm = _make_spmm(n, e, 4000)
    segsum = _make_segsum(n, t)
    gatk = _make_gat(n, t)

    p1, p2, ptt, pmol = (params['gcn1'], params['gcn2'], params['tt'],
                         params['mol'])

    rc = row * 16384 + col
    ax1 = spmm(x0_T, rc, adj_value)
    x1 = _gcn_update(ax1, p1['W'], _col(p1['b']), _col(p1['bn_g']),
                     _col(p1['bn_b']))
    ax2 = spmm(x1, rc, adj_value)
    xb0, hs, asrc = _post_gcn2(
        ax2, p2['W'], _col(p2['b']), _col(p2['bn_g']), _col(p2['bn_b']),
        _col(ptt['bn0_g']), _col(ptt['bn0_b']), ptt['Ws'], _col(ptt['a_s']))
    segx = segsum(xb0, tt_node_batch)
    meanb, adst = _mid_pool(segx, _col(ptt['bn1_g']), _col(ptt['bn1_b']),
                            ptt['Wd'], _col(ptt['a_d']))
    u, den = gatk(hs, asrc.reshape(n), adst.reshape(t), tt_node_batch)
    mol_in = {k: (_col(v) if v.ndim == 1 else v) for k, v in pmol.items()}
    pred_in = {k: (_col(v) if v.ndim == 1 else v)
               for k, v in params['pred'].items()}
    y = _final(u, den.reshape(1, t), meanb, _col(ptt['bias']), ptt['Wih'],
               ptt['Whh'], _col(ptt['bih']), _col(ptt['bhh']),
               _col(ptt['bn2_g']), _col(ptt['bn2_b']), mol_in,
               tt_graph_batch.reshape(1, t), pred_in)
    return y.reshape(m, 1)


# speedup vs baseline: 1.0067x; 1.0067x over previous
import functools

import jax
import jax.numpy as jnp
from jax import lax
from jax.experimental import pallas as pl
from jax.experimental.pallas import tpu as pltpu
from jax.experimental.pallas import tpu_sc as plsc

EPS = 1e-5
M_SEG = 500
NC = 2
NS = 16
NW = NC * NS
CPL = 128 // NW

_SC_PARAMS = pltpu.CompilerParams(needs_layout_passes=False)


def _mesh():
    return plsc.VectorSubcoreMesh(core_axis_name="c", subcore_axis_name="s")


def _wid():
    return lax.axis_index("s") * NC + lax.axis_index("c")


def _zero_acc(acc, nwords):
    z = jnp.zeros((16,), jnp.float32)

    @plsc.parallel_loop(0, nwords // 16, unroll=5)
    def _(i):
        for cc in range(CPL):
            acc[cc, pl.ds(i * 16, 16)] = z


def _make_spmm(n, e, chunk):
    nchunk = e // chunk
    groups = chunk // 16

    @functools.partial(
        pl.kernel,
        out_type=jax.ShapeDtypeStruct((128, n), jnp.float32),
        mesh=_mesh(),
        scratch_types=[
            pltpu.VMEM((CPL, n), jnp.float32),
            pltpu.VMEM((CPL, n), jnp.float32),
            pltpu.VMEM((chunk,), jnp.int32),
            pltpu.VMEM((chunk,), jnp.int32),
            pltpu.VMEM((chunk,), jnp.float32),
            pltpu.VMEM((chunk,), jnp.float32),
            pltpu.SemaphoreType.DMA,
            pltpu.SemaphoreType.DMA,
        ],
        compiler_params=_SC_PARAMS,
    )
    def spmm(xt_hbm, rc_hbm, val_hbm, out_hbm,
             xv, acc, rcb0, rcb1, valb0, valb1, sem0, sem1):
        w = _wid()
        pltpu.async_copy(xt_hbm.at[pl.ds(w * CPL, CPL)], xv, sem0)
        _zero_acc(acc, n)
        pltpu.make_async_copy(xt_hbm.at[pl.ds(0, CPL)], xv, sem0).wait()
        sems = (sem0, sem1)
        rcbs, valbs = (rcb0, rcb1), (valb0, valb1)

        def start(g, slot):
            s = sems[slot]
            pltpu.async_copy(rc_hbm.at[pl.ds(g * chunk, chunk)], rcbs[slot], s)
            pltpu.async_copy(val_hbm.at[pl.ds(g * chunk, chunk)], valbs[slot], s)

        def wait(slot):
            s = sems[slot]
            pltpu.make_async_copy(rc_hbm.at[pl.ds(0, chunk)], rcbs[slot], s).wait()
            pltpu.make_async_copy(val_hbm.at[pl.ds(0, chunk)], valbs[slot], s).wait()

        def process(slot):
            rcb, vb = rcbs[slot], valbs[slot]

            @plsc.parallel_loop(0, groups, unroll=10)
            def _(i):
                ds = pl.ds(i * 16, 16)
                rc = rcb[ds]
                rvec = lax.shift_right_logical(rc, 14)
                cvec = rc & 16383
                vvec = vb[ds]
                for cc in range(CPL):
                    ci = jnp.full((16,), cc, jnp.int32)
                    g = plsc.load_gather(xv, [ci, cvec])
                    plsc.addupdate_scatter(acc, [ci, rvec], g * vvec)

        start(0, 0)

        def pair(j, _):
            g0 = j * 2
            start(g0 + 1, 1)
            wait(0)
            process(0)

            @pl.when(g0 + 2 < nchunk)
            def _():
                start(g0 + 2, 0)

            wait(1)
            process(1)
            return 0

        lax.fori_loop(0, nchunk // 2, pair, 0)
        pltpu.sync_copy(acc, out_hbm.at[pl.ds(w * CPL, CPL)])

    return spmm


def _make_segsum(n, t):
    @functools.partial(
        pl.kernel,
        out_type=jax.ShapeDtypeStruct((128, t), jnp.float32),
        mesh=_mesh(),
        scratch_types=[
            pltpu.VMEM((CPL, n), jnp.float32),
            pltpu.VMEM((n,), jnp.int32),
            pltpu.VMEM((CPL, t), jnp.float32),
            pltpu.SemaphoreType.DMA,
        ],
        compiler_params=_SC_PARAMS,
    )
    def seg(x_hbm, batch_hbm, out_hbm, xv, bv, acc, sem):
        w = _wid()
        pltpu.async_copy(x_hbm.at[pl.ds(w * CPL, CPL)], xv, sem)
        pltpu.async_copy(batch_hbm, bv, sem)
        _zero_acc(acc, t)
        pltpu.make_async_copy(x_hbm.at[pl.ds(0, CPL)], xv, sem).wait()
        pltpu.make_async_copy(batch_hbm, bv, sem).wait()

        @plsc.parallel_loop(0, n // 16, unroll=8)
        def _(i):
            ds = pl.ds(i * 16, 16)
            bvec = bv[ds]
            for cc in range(CPL):
                ci = jnp.full((16,), cc, jnp.int32)
                plsc.addupdate_scatter(acc, [ci, bvec], xv[cc, ds])
        pltpu.sync_copy(acc, out_hbm.at[pl.ds(w * CPL, CPL)])

    return seg


def _make_gat(n, t):
    @functools.partial(
        pl.kernel,
        out_type=(jax.ShapeDtypeStruct((128, t), jnp.float32),
                  jax.ShapeDtypeStruct((t,), jnp.float32)),
        mesh=_mesh(),
        scratch_types=[
            pltpu.VMEM((CPL, n), jnp.float32),
            pltpu.VMEM((n,), jnp.float32),
            pltpu.VMEM((t,), jnp.float32),
            pltpu.VMEM((n,), jnp.int32),
            pltpu.VMEM((CPL, t), jnp.float32),
            pltpu.VMEM((t,), jnp.float32),
            pltpu.SemaphoreType.DMA,
        ],
        compiler_params=_SC_PARAMS,
    )
    def gat(hs_hbm, asrc_hbm, adst_hbm, batch_hbm, u_hbm, den_hbm,
            xv, asv, adv, bv, accu, accd, sem):
        w = _wid()
        pltpu.async_copy(hs_hbm.at[pl.ds(w * CPL, CPL)], xv, sem)
        pltpu.async_copy(asrc_hbm, asv, sem)
        pltpu.async_copy(adst_hbm, adv, sem)
        pltpu.async_copy(batch_hbm, bv, sem)
        _zero_acc(accu, t)
        z = jnp.zeros((16,), jnp.float32)

        @plsc.parallel_loop(0, t // 16, unroll=5)
        def _(i):
            accd[pl.ds(i * 16, 16)] = z

        pltpu.make_async_copy(hs_hbm.at[pl.ds(0, CPL)], xv, sem).wait()
        pltpu.make_async_copy(asrc_hbm, asv, sem).wait()
        pltpu.make_async_copy(adst_hbm, adv, sem).wait()
        pltpu.make_async_copy(batch_hbm, bv, sem).wait()

        def edge_pass(with_den):
            @plsc.parallel_loop(0, n // 16, unroll=8)
            def _(i):
                ds = pl.ds(i * 16, 16)
                bvec = bv[ds]
                ev = asv[ds] + plsc.load_gather(adv, [bvec])
                ev = jnp.where(ev > 0, ev, 0.01 * ev)
                ex = jnp.exp(ev)
                if with_den:
                    plsc.addupdate_scatter(accd, [bvec], ex)
                for cc in range(CPL):
                    ci = jnp.full((16,), cc, jnp.int32)
                    plsc.addupdate_scatter(accu, [ci, bvec], xv[cc, ds] * ex)

        @pl.when(w == 0)
        def _():
            edge_pass(True)

        @pl.when(w != 0)
        def _():
            edge_pass(False)

        pltpu.sync_copy(accu, u_hbm.at[pl.ds(w * CPL, CPL)])

        @pl.when(w == 0)
        def _():
            pltpu.sync_copy(accd, den_hbm)

    return gat


def _bn_t(h, g, b):
    mu = jnp.mean(h, axis=1, keepdims=True)
    var = jnp.mean((h - mu) ** 2, axis=1, keepdims=True)
    return (h - mu) * lax.rsqrt(var + EPS) * g + b


def _mmT(w, x):
    return lax.dot_general(w, x, (((0,), (0,)), ((), ())),
                           preferred_element_type=jnp.float32)


def _elu(x):
    return jnp.where(x > 0, x, jnp.exp(jnp.where(x > 0, 0.0, x)) - 1.0)


def _gru_t(x, h, wih, whh, bih, bhh):
    gi = _mmT(wih, x) + bih
    gh = _mmT(whh, h) + bhh
    r = jax.nn.sigmoid(gi[0:128] + gh[0:128])
    zz = jax.nn.sigmoid(gi[128:256] + gh[128:256])
    nn_ = jnp.tanh(gi[256:384] + r * gh[256:384])
    return (1.0 - zz) * nn_ + zz * h


def _gcn_update(ax_T, w, b, g, bb):
    n = ax_T.shape[1]

    def body(ax_ref, w_ref, b_ref, g_ref, bb_ref, o_ref):
        h = jnp.maximum(_mmT(w_ref[...], ax_ref[...]) + b_ref[...], 0.0)
        o_ref[...] = _bn_t(h, g_ref[...], bb_ref[...])

    return pl.pallas_call(
        body, out_shape=jax.ShapeDtypeStruct((128, n), jnp.float32),
    )(ax_T, w, b, g, bb)


def _post_gcn2(ax_T, w, b, g, bb, g0, b0, ws, a_s):
    n = ax_T.shape[1]

    def body(ax_ref, w_ref, b_ref, g_ref, bb_ref, g0_ref, b0_ref, ws_ref,
             as_ref, xb0_ref, hs_ref, asrc_ref):
        h = jnp.maximum(_mmT(w_ref[...], ax_ref[...]) + b_ref[...], 0.0)
        x2 = _bn_t(h, g_ref[...], bb_ref[...])
        xb0 = _bn_t(x2, g0_ref[...], b0_ref[...])
        hs = _mmT(ws_ref[...], xb0)
        xb0_ref[...] = xb0
        hs_ref[...] = hs
        asrc_ref[...] = jnp.sum(hs * as_ref[...], axis=0, keepdims=True)

    return pl.pallas_call(
        body,
        out_shape=(jax.ShapeDtypeStruct((128, n), jnp.float32),
                   jax.ShapeDtypeStruct((128, n), jnp.float32),
                   jax.ShapeDtypeStruct((1, n), jnp.float32)),
    )(ax_T, w, b, g, bb, g0, b0, ws, a_s)


def _mid_pool(segx_T, g1, b1, wd, a_d):
    t = segx_T.shape[1]

    def body(sx_ref, g1_ref, b1_ref, wd_ref, ad_ref, mb_ref, adst_ref):
        mean = jnp.maximum(sx_ref[...], 0.0)
        mb = _bn_t(mean, g1_ref[...], b1_ref[...])
        hd = _mmT(wd_ref[...], mb)
        mb_ref[...] = mb
        adst_ref[...] = jnp.sum(hd * ad_ref[...], axis=0, keepdims=True)

    return pl.pallas_call(
        body,
        out_shape=(jax.ShapeDtypeStruct((128, t), jnp.float32),
                   jax.ShapeDtypeStruct((1, t), jnp.float32)),
    )(segx_T, g1, b1, wd, a_d)


def _final(u_T, den, meanb_T, bias_tt, wih_tt, whh_tt, bih_tt, bhh_tt,
           g2_tt, b2_tt, mol, batch2, pred):
    t = u_T.shape[1]
    m = M_SEG

    def body(u_ref, den_ref, mb_ref, bias_ref, wih_ref, whh_ref, bih_ref,
             bhh_ref, g2_ref, b2_ref,
             g0m_ref, b0m_ref, g1m_ref, b1m_ref, g2m_ref, b2m_ref,
             wsm_ref, wdm_ref, asm_ref, adm_ref, biasm_ref,
             wihm_ref, whhm_ref, bihm_ref, bhhm_ref,
             batch_ref, w1_ref, b1p_ref, w2_ref, b2p_ref, o_ref):
        meanb = mb_ref[...]
        gat = u_ref[...] / (den_ref[...] + 1e-16) + bias_ref[...]
        g = _elu(gat)
        gg = _gru_t(g, meanb, wih_ref[...], whh_ref[...], bih_ref[...],
                    bhh_ref[...])
        tt = _bn_t(jnp.maximum(gg, 0.0), g2_ref[...], b2_ref[...])
        xb = _bn_t(tt, g0m_ref[...], b0m_ref[...])
        sel = (lax.broadcasted_iota(jnp.int32, (m, t), 0) ==
               jnp.broadcast_to(batch_ref[...], (m, t))).astype(jnp.float32)
        segx = lax.dot_general(xb, sel, (((1,), (1,)), ((), ())),
                               preferred_element_type=jnp.float32)
        meanm = jnp.maximum(segx, 0.0)
        meanmb = _bn_t(meanm, g1m_ref[...], b1m_ref[...])
        hs = _mmT(wsm_ref[...], xb)
        asrc = jnp.sum(hs * asm_ref[...], axis=0, keepdims=True)
        hd = _mmT(wdm_ref[...], meanmb)
        adst = jnp.sum(hd * adm_ref[...], axis=0, keepdims=True)
        ev = asrc + lax.dot_general(adst, sel, (((1,), (0,)), ((), ())),
                                    preferred_element_type=jnp.float32)
        ev = jnp.where(ev > 0, ev, 0.01 * ev)
        ex = jnp.exp(ev)
        denm = lax.dot_general(ex, sel, (((1,), (1,)), ((), ())),
                               preferred_element_type=jnp.float32)
        um = lax.dot_general(hs * ex, sel, (((1,), (1,)), ((), ())),
                             preferred_element_type=jnp.float32)
        gatm = um / (denm + 1e-16) + biasm_ref[...]
        gm = _elu(gatm)
        ggm = _gru_t(gm, meanmb, wihm_ref[...], whhm_ref[...], bihm_ref[...],
                     bhhm_ref[...])
        molv = _bn_t(jnp.maximum(ggm, 0.0), g2m_ref[...], b2m_ref[...])
        h1 = jnp.maximum(_mmT(w1_ref[...], molv) + b1p_ref[...], 0.0)
        o_ref[...] = _mmT(w2_ref[...], h1) + b2p_ref[...]

    return pl.pallas_call(
        body, out_shape=jax.ShapeDtypeStruct((1, m), jnp.float32),
    )(u_T, den, meanb_T, bias_tt, wih_tt, whh_tt, bih_tt, bhh_tt, g2_tt,
      b2_tt, mol['bn0_g'], mol['bn0_b'], mol['bn1_g'], mol['bn1_b'],
      mol['bn2_g'], mol['bn2_b'], mol['Ws'], mol['Wd'], mol['a_s'],
      mol['a_d'], mol['bias'], mol['Wih'], mol['Whh'], mol['bih'],
      mol['bhh'], batch2, pred['W1'], pred['b1'], pred['W2'], pred['b2'])


def _col(v):
    return v.reshape(-1, 1)


def kernel(node_attr, adj_index, adj_value, tt_node_batch, tt_graph_batch,
           params):
    n, d = node_attr.shape
    e = adj_value.shape[0]
    t = tt_graph_batch.shape[0]
    m = M_SEG

    row = adj_index[0]
    col = adj_index[1]
    x0_T = jnp.swapaxes(node_attr, 0, 1)

    spmm = _make_spmm(n, e, 8000)
    segsum = _make_segsum(n, t)
    gatk = _make_gat(n, t)

    p1, p2, ptt, pmol = (params['gcn1'], params['gcn2'], params['tt'],
                         params['mol'])

    rc = row * 16384 + col
    ax1 = spmm(x0_T, rc, adj_value)
    x1 = _gcn_update(ax1, p1['W'], _col(p1['b']), _col(p1['bn_g']),
                     _col(p1['bn_b']))
    ax2 = spmm(x1, rc, adj_value)
    xb0, hs, asrc = _post_gcn2(
        ax2, p2['W'], _col(p2['b']), _col(p2['bn_g']), _col(p2['bn_b']),
        _col(ptt['bn0_g']), _col(ptt['bn0_b']), ptt['Ws'], _col(ptt['a_s']))
    segx = segsum(xb0, tt_node_batch)
    meanb, adst = _mid_pool(segx, _col(ptt['bn1_g']), _col(ptt['bn1_b']),
                            ptt['Wd'], _col(ptt['a_d']))
    u, den = gatk(hs, asrc.reshape(n), adst.reshape(t), tt_node_batch)
    mol_in = {k: (_col(v) if v.ndim == 1 else v) for k, v in pmol.items()}
    pred_in = {k: (_col(v) if v.ndim == 1 else v)
               for k, v in params['pred'].items()}
    y = _final(u, den.reshape(1, t), meanb, _col(ptt['bias']), ptt['Wih'],
               ptt['Whh'], _col(ptt['bih']), _col(ptt['bhh']),
               _col(ptt['bn2_g']), _col(ptt['bn2_b']), mol_in,
               tt_graph_batch.reshape(1, t), pred_in)
    return y.reshape(m, 1)
